# baseline (speedup 1.0000x reference)
import jax
import jax.numpy as jnp
from jax import lax
from jax.experimental import pallas as pl
from jax.experimental.pallas import tpu as pltpu
from jax.experimental.pallas import tpu_sc as plsc

_SCORE_THRESH = 0.25
_NMS_THRESH = 0.5
_MAX_DET = 300
_NEG = -1e9
_BIG = 1e9
_N = 5000
_SPAD = 5008
_NCHUNK = _SPAD // 16


def _sc_nms(x1_hbm, y1_hbm, x2_hbm, y2_hbm, sc_hbm, out_hbm,
            x1_v, y1_v, x2_v, y2_v, wk_v, ar_v, rows_v):
    c = lax.axis_index("c")
    s = lax.axis_index("s")
    b = 2 * c + s // 8
    active = lax.rem(s, 8) == 0

    pltpu.sync_copy(x1_hbm.at[b], x1_v)
    pltpu.sync_copy(y1_hbm.at[b], y1_v)
    pltpu.sync_copy(x2_hbm.at[b], x2_v)
    pltpu.sync_copy(y2_hbm.at[b], y2_v)
    pltpu.sync_copy(sc_hbm.at[b], wk_v)

    lanes = lax.iota(jnp.int32, 16)
    lanef = lanes.astype(jnp.float32)

    def init_chunk(j, carry):
        dsl = pl.ds(j * 16, 16)
        scv = wk_v[dsl]
        wk_v[dsl] = jnp.where(scv > _SCORE_THRESH, scv, _NEG)
        ar_v[dsl] = (x2_v[dsl] - x1_v[dsl]) * (y2_v[dsl] - y1_v[dsl])
        return carry

    lax.fori_loop(0, _NCHUNK, init_chunk, 0)

    def step(t, carry):
        def amax_chunk(j, mc):
            m, mi = mc
            dsl = pl.ds(j * 16, 16)
            w = wk_v[dsl]
            gi = lax.convert_element_type(j * 16, jnp.float32) + lanef
            bm = w > m
            return jnp.where(bm, w, m), jnp.where(bm, gi, mi)

        m0 = jnp.full((16,), _NEG, jnp.float32)
        i0 = jnp.full((16,), _BIG, jnp.float32)
        m, mi = lax.fori_loop(0, _NCHUNK, amax_chunk, (m0, i0))
        lm = jnp.max(m)
        li = jnp.min(jnp.where(m == lm, mi, _BIG))
        valid = lm > _NEG * 0.5
        kloc = jnp.clip(li, 0.0, float(_N - 1)).astype(jnp.int32)
        kvec = jnp.full((16,), 0, jnp.int32) + kloc

        bx1 = plsc.load_gather(x1_v, [kvec])
        by1 = plsc.load_gather(y1_v, [kvec])
        bx2 = plsc.load_gather(x2_v, [kvec])
        by2 = plsc.load_gather(y2_v, [kvec])
        barea = (bx2 - bx1) * (by2 - by1)

        def iou_chunk(j, carry2):
            dsl = pl.ds(j * 16, 16)
            w = wk_v[dsl]
            xx1 = jnp.maximum(x1_v[dsl], bx1)
            yy1 = jnp.maximum(y1_v[dsl], by1)
            xx2 = jnp.minimum(x2_v[dsl], bx2)
            yy2 = jnp.minimum(y2_v[dsl], by2)
            inter = (jnp.maximum(xx2 - xx1, 0.0)
                     * jnp.maximum(yy2 - yy1, 0.0))
            iou = inter / (barea + ar_v[dsl] - inter + 1e-9)
            gi = lax.convert_element_type(j * 16, jnp.float32) + lanef
            kill = ((iou > _NMS_THRESH) & valid) | (gi == li)
            wk_v[dsl] = jnp.where(kill, _NEG, w)
            return carry2

        lax.fori_loop(0, _NCHUNK, iou_chunk, 0)

        vf = jnp.where(valid, 1.0, 0.0)
        row = vf * (jnp.where(lanes == 0, bx1, 0.0)
                    + jnp.where(lanes == 1, by1, 0.0)
                    + jnp.where(lanes == 2, bx2, 0.0)
                    + jnp.where(lanes == 3, by2, 0.0)
                    + jnp.where(lanes == 4, lm, 0.0))
        plsc.store_scatter(
            rows_v, [jnp.full((16,), 0, jnp.int32) + t, lanes], row)
        return carry

    lax.fori_loop(0, _MAX_DET, step, 0)

    @pl.when(active)
    def _():
        pltpu.sync_copy(rows_v, out_hbm.at[b])


def kernel(boxes, scores):
    bsz, n, _ = boxes.shape
    padw = ((0, 0), (0, _SPAD - n))
    x1 = jnp.pad(boxes[:, :, 0], padw)
    y1 = jnp.pad(boxes[:, :, 1], padw)
    x2 = jnp.pad(boxes[:, :, 2], padw)
    y2 = jnp.pad(boxes[:, :, 3], padw)
    sc = jnp.pad(scores, padw, constant_values=-1.0)

    mesh = plsc.VectorSubcoreMesh(core_axis_name="c", subcore_axis_name="s",
                                  num_cores=2, num_subcores=16)
    f = pl.kernel(
        _sc_nms,
        out_type=jax.ShapeDtypeStruct((bsz, _MAX_DET, 16), jnp.float32),
        mesh=mesh,
        compiler_params=pltpu.CompilerParams(needs_layout_passes=False),
        scratch_types=[
            pltpu.VMEM((_SPAD,), jnp.float32),
            pltpu.VMEM((_SPAD,), jnp.float32),
            pltpu.VMEM((_SPAD,), jnp.float32),
            pltpu.VMEM((_SPAD,), jnp.float32),
            pltpu.VMEM((_SPAD,), jnp.float32),
            pltpu.VMEM((_SPAD,), jnp.float32),
            pltpu.VMEM((_MAX_DET, 16), jnp.float32),
        ],
    )
    out = f(x1, y1, x2, y2, sc)
    return out[:, :, :5]
